# baseline (speedup 1.0000x reference)
import functools

import jax
import jax.numpy as jnp
from jax import lax
from jax.experimental import pallas as pl

N_TOKENS = 16384
D_MODEL = 2048
N_EXPERTS = 64
BLOCK_M = 512


def _router_body(x_ref, w_ref, probs_ref, idx_ref):
    x = x_ref[...]
    w = w_ref[...]
    logits = lax.dot_general(
        x, w, (((1,), (1,)), ((), ())), preferred_element_type=jnp.float32
    )
    col = lax.broadcasted_iota(jnp.int32, logits.shape, 1)

    m1 = jnp.max(logits, axis=1, keepdims=True)
    is1 = logits == m1
    i1 = jnp.min(jnp.where(is1, col, N_EXPERTS), axis=1, keepdims=True)

    masked = jnp.where(col == i1, -jnp.inf, logits)
    m2 = jnp.max(masked, axis=1, keepdims=True)
    is2 = masked == m2
    i2 = jnp.min(jnp.where(is2, col, N_EXPERTS), axis=1, keepdims=True)

    v1 = 1.0 / (1.0 + jnp.exp(m2 - m1))
    v2 = 1.0 - v1

    probs_ref[...] = jnp.where(
        col == i1, v1, jnp.where(col == i2, v2, jnp.float32(0.0))
    )
    idx_ref[...] = jnp.concatenate([i1, i2], axis=1)


@jax.jit
def kernel(x, W):
    grid = (N_TOKENS // BLOCK_M,)
    probs, idx = pl.pallas_call(
        _router_body,
        grid=grid,
        in_specs=[
            pl.BlockSpec((BLOCK_M, D_MODEL), lambda i: (i, 0)),
            pl.BlockSpec((N_EXPERTS, D_MODEL), lambda i: (0, 0)),
        ],
        out_specs=[
            pl.BlockSpec((BLOCK_M, N_EXPERTS), lambda i: (i, 0)),
            pl.BlockSpec((BLOCK_M, 2), lambda i: (i, 0)),
        ],
        out_shape=[
            jax.ShapeDtypeStruct((N_TOKENS, N_EXPERTS), jnp.float32),
            jax.ShapeDtypeStruct((N_TOKENS, 2), jnp.int32),
        ],
    )(x, W)
    return probs, idx


# speedup vs baseline: 1.2104x; 1.2104x over previous
import functools

import jax
import jax.numpy as jnp
from jax import lax
from jax.experimental import pallas as pl

N_TOKENS = 16384
D_MODEL = 2048
N_EXPERTS = 64
BLOCK_M = 2048


def _router_body(x_ref, w_ref, probs_ref, idx_ref):
    x = x_ref[...]
    w = w_ref[...]
    logits = lax.dot_general(
        x, w, (((1,), (1,)), ((), ())), preferred_element_type=jnp.float32
    )
    col = lax.broadcasted_iota(jnp.int32, logits.shape, 1)

    m1 = jnp.max(logits, axis=1, keepdims=True)
    is1 = logits == m1
    i1 = jnp.min(jnp.where(is1, col, N_EXPERTS), axis=1, keepdims=True)

    masked = jnp.where(col == i1, -jnp.inf, logits)
    m2 = jnp.max(masked, axis=1, keepdims=True)
    is2 = masked == m2
    i2 = jnp.min(jnp.where(is2, col, N_EXPERTS), axis=1, keepdims=True)

    v1 = 1.0 / (1.0 + jnp.exp(m2 - m1))
    v2 = 1.0 - v1

    probs_ref[...] = jnp.where(
        col == i1, v1, jnp.where(col == i2, v2, jnp.float32(0.0))
    )
    idx_ref[...] = jnp.concatenate([i1, i2], axis=1)


@jax.jit
def kernel(x, W):
    grid = (N_TOKENS // BLOCK_M,)
    probs, idx = pl.pallas_call(
        _router_body,
        grid=grid,
        in_specs=[
            pl.BlockSpec((BLOCK_M, D_MODEL), lambda i: (i, 0)),
            pl.BlockSpec((N_EXPERTS, D_MODEL), lambda i: (0, 0)),
        ],
        out_specs=[
            pl.BlockSpec((BLOCK_M, N_EXPERTS), lambda i: (i, 0)),
            pl.BlockSpec((BLOCK_M, 2), lambda i: (i, 0)),
        ],
        out_shape=[
            jax.ShapeDtypeStruct((N_TOKENS, N_EXPERTS), jnp.float32),
            jax.ShapeDtypeStruct((N_TOKENS, 2), jnp.int32),
        ],
    )(x, W)
    return probs, idx
